# baseline (speedup 1.0000x reference)
import functools

import numpy as np
import jax
import jax.numpy as jnp
from jax import lax
from jax.experimental import pallas as pl
from jax.experimental.pallas import tpu as pltpu
from jax.experimental.pallas import tpu_sc as plsc

_B = 8
_N = 100000
_S = 50000
_D = 128
_NW = 32
_ROWS = _B * _S

_C = 128
_K = 2
_NCHUNKS = _ROWS // _C
_CPW = -(-_NCHUNKS // _NW)
_LAST_COUNT = _NCHUNKS - (_NW - 1) * _CPW
_NG = -(-_CPW // _K)

_XC = 128
_XG = 32
_XE = _ROWS * 3
_XNCH = _XE // _XC
_XCPW = -(-_XNCH // _NW)
_XLAST_COUNT = _XNCH - (_NW - 1) * _XCPW
_NSLOTS = _NG + 1
_NPAIR = -(-_NSLOTS // 2)


def _threefry2x32(k1, k2, x1, x2):
    rot = [[13, 15, 26, 6], [17, 29, 16, 24]]
    ks = [k1, k2, (k1 ^ k2 ^ np.uint32(0x1BD11BDA))]
    x1 = (x1 + ks[0]).astype(np.uint32)
    x2 = (x2 + ks[1]).astype(np.uint32)
    for i in range(5):
        for r in rot[i % 2]:
            x1 = (x1 + x2).astype(np.uint32)
            x2 = ((x2 << np.uint32(r)) | (x2 >> np.uint32(32 - r)))
            x2 = x2 ^ x1
        x1 = (x1 + ks[(i + 1) % 3]).astype(np.uint32)
        x2 = (x2 + ks[(i + 2) % 3] + np.uint32(i + 1)).astype(np.uint32)
    return x1, x2


def _iota_2x32(n):
    c = np.arange(n, dtype=np.uint64)
    return ((c >> np.uint64(32)).astype(np.uint32),
            (c & np.uint64(0xFFFFFFFF)).astype(np.uint32))


def _random_bits32(key, n):
    c1, c2 = _iota_2x32(n)
    b1, b2 = _threefry2x32(key[0], key[1], c1, c2)
    return b1 ^ b2


def _split2(key):
    c1, c2 = _iota_2x32(2)
    b1, b2 = _threefry2x32(key[0], key[1], c1, c2)
    return np.stack([b1, b2], axis=1)


def _permutation(seed, n):
    key = np.array([seed >> 32, seed & 0xFFFFFFFF], dtype=np.uint32)
    x = np.arange(n, dtype=np.int32)
    num_rounds = int(np.ceil(3 * np.log(max(1, n)) / np.log(2**32 - 1)))
    for _ in range(num_rounds):
        ks = _split2(key)
        key, subkey = ks[0], ks[1]
        sort_keys = _random_bits32(subkey, n)
        x = x[np.argsort(sort_keys, kind="stable")]
    return x


_SAMPLE_IDX = _permutation(42, _N)[:_S]

_GIDX = np.zeros((_NW * _CPW, _C), dtype=np.int32)
_GIDX[:_NCHUNKS] = (
    (np.arange(_B, dtype=np.int32) * _N)[:, None] + _SAMPLE_IDX[None, :]
).reshape(_NCHUNKS, _C)
_GIDX = _GIDX.reshape(_NW, _CPW, _C)

_P = np.arange(_XE, dtype=np.int64)
_XIDX = np.zeros((_NW * _XCPW, _XC), dtype=np.int32)
_XIDX[:_XNCH] = (
    (_P // (_B * _S)) * (_B * _N) + ((_P % (_B * _S)) // _S) * _N
    + _SAMPLE_IDX[_P % _S]
).astype(np.int32).reshape(_XNCH, _XC)
_XIDX = _XIDX.reshape(_NW, _XCPW, _XC)
del _P

_mesh = plsc.VectorSubcoreMesh(core_axis_name="c", subcore_axis_name="s")


@functools.partial(
    pl.kernel,
    out_type=[
        jax.ShapeDtypeStruct((_ROWS, _D), jnp.float32),
        jax.ShapeDtypeStruct((_XE,), jnp.float32),
    ],
    mesh=_mesh,
    scratch_types=[
        pltpu.VMEM((_CPW, _C), jnp.int32),
        pltpu.VMEM((_XCPW, _XC), jnp.int32),
        pltpu.VMEM((2, _K, _C, _D), jnp.float32),
        pltpu.VMEM((_XG * _XC,), jnp.float32),
        pltpu.VMEM((_XG * _XC,), jnp.float32),
        pltpu.SemaphoreType.DMA((2,)),
        pltpu.SemaphoreType.DMA((2,)),
        pltpu.SemaphoreType.DMA((2,)),
        pltpu.SemaphoreType.DMA((2,)),
    ],
)
def _gather(feat_hbm, xyz_hbm, gidx_hbm, xidx_hbm, feat_out, xyz_out,
            idx_v, xidx_v, fbuf, xbuf0, xbuf1, gsem, wsem, xgsem, xwsem):
    xbufs = (xbuf0, xbuf1)
    wid = lax.axis_index("s") * 2 + lax.axis_index("c")
    count = jnp.where(wid == _NW - 1, _LAST_COUNT, _CPW)
    xcount = jnp.where(wid == _NW - 1, _XLAST_COUNT, _XCPW)
    fobase = wid * _CPW
    xobase = wid * _XCPW * _XC
    pltpu.sync_copy(gidx_hbm.at[wid], idx_v)
    pltpu.sync_copy(xidx_hbm.at[wid], xidx_v)

    def fg_copy(t, bank, b):
        return pltpu.make_async_copy(
            feat_hbm.at[idx_v.at[t]], fbuf.at[bank].at[b], gsem.at[bank])

    def fw_copy(t, bank, b):
        return pltpu.make_async_copy(
            fbuf.at[bank].at[b],
            feat_out.at[pl.ds((fobase + t) * _C, _C)], wsem.at[bank])

    def xg_copy(xt, bank, b):
        return pltpu.make_async_copy(
            xyz_hbm.at[xidx_v.at[xt]],
            xbufs[bank].at[pl.ds(b * _XC, _XC)], xgsem.at[bank])

    def fire_fgathers(g, bank):
        for b in range(_K):
            t = g * _K + b

            @pl.when(t < count)
            def _():
                fg_copy(t, bank, b).start()

    def fire_xgathers(g, bank):
        for b in range(_XG):
            xt = g * _XG + b

            @pl.when(xt < xcount)
            def _():
                xg_copy(xt, bank, b).start()

    def xw_full(g, bank):
        return pltpu.make_async_copy(
            xbufs[bank],
            xyz_out.at[pl.ds(xobase + g * _XG * _XC, _XG * _XC)],
            xwsem.at[bank])

    def xw_part(xt, bank, b):
        return pltpu.make_async_copy(
            xbufs[bank].at[pl.ds(b * _XC, _XC)],
            xyz_out.at[pl.ds(xobase + xt * _XC, _XC)], xwsem.at[bank])

    def slot(g, p):
        q = 1 - p
        for b in range(_K):
            t = g * _K + b

            @pl.when(t < count)
            def _():
                fg_copy(t, p, b).wait()
        for b in range(_K):
            t = g * _K + b

            @pl.when(t < count)
            def _():
                fw_copy(t, p, b).start()
        for b in range(_K):
            tp = (g - 1) * _K + b

            @pl.when((g >= 1) & (tp < count))
            def _():
                fw_copy(tp, q, b).wait()
        fire_fgathers(g + 1, q)

        for b in range(_XG):
            xt = g * _XG + b

            @pl.when(xt < xcount)
            def _():
                xg_copy(xt, p, b).wait()
        g_full = (g * _XG + _XG) <= xcount
        g_any = (g * _XG) < xcount

        @pl.when(g_any & g_full)
        def _():
            xw_full(g, p).start()
        for b in range(_XG):
            xt = g * _XG + b

            @pl.when((~g_full) & (xt < xcount))
            def _():
                xw_part(xt, p, b).start()
        gp_full = ((g - 1) * _XG + _XG) <= xcount
        gp_any = (g >= 1) & (((g - 1) * _XG) < xcount)

        @pl.when(gp_any & gp_full)
        def _():
            xw_full(g - 1, q).wait()
        for b in range(_XG):
            xt = (g - 1) * _XG + b

            @pl.when(gp_any & (~gp_full) & (xt < xcount))
            def _():
                xw_part(xt, q, b).wait()
        fire_xgathers(g + 1, q)

    fire_fgathers(0, 0)
    fire_xgathers(0, 0)

    def pair(m, carry):
        slot(2 * m, 0)
        slot(2 * m + 1, 1)
        return carry

    lax.fori_loop(0, _NPAIR, pair, 0)


def kernel(xyz, features):
    feat_out, xyz_out = _gather(
        features.reshape(_B * _N, _D),
        jnp.transpose(xyz, (2, 0, 1)).reshape(_B * _N * 3),
        jnp.asarray(_GIDX),
        jnp.asarray(_XIDX),
    )
    new_xyz = jnp.transpose(xyz_out.reshape(3, _B, _S), (1, 2, 0))
    new_features = feat_out.reshape(_B, _S, _D)
    sample_idx_b = jnp.broadcast_to(jnp.asarray(_SAMPLE_IDX), (_B, _S))
    return (new_xyz, new_features, sample_idx_b)


# speedup vs baseline: 1.0875x; 1.0875x over previous
import functools

import numpy as np
import jax
import jax.numpy as jnp
from jax import lax
from jax.experimental import pallas as pl
from jax.experimental.pallas import tpu as pltpu
from jax.experimental.pallas import tpu_sc as plsc

_B = 8
_N = 100000
_S = 50000
_D = 128
_NW = 32
_ROWS = _B * _S

_C = 128
_K = 2
_NCHUNKS = _ROWS // _C
_CPW = -(-_NCHUNKS // _NW)
_LAST_COUNT = _NCHUNKS - (_NW - 1) * _CPW
_NG = -(-_CPW // _K)

_XC = 128
_XG = 8
_XE = _ROWS * 3
_XNCH = _XE // _XC
_XCPW = -(-_XNCH // _NW)
_XLAST_COUNT = _XNCH - (_NW - 1) * _XCPW
_NSLOTS = _NG + 1
_NPAIR = -(-_NSLOTS // 2)


def _threefry2x32(k1, k2, x1, x2):
    rot = [[13, 15, 26, 6], [17, 29, 16, 24]]
    ks = [k1, k2, (k1 ^ k2 ^ np.uint32(0x1BD11BDA))]
    x1 = (x1 + ks[0]).astype(np.uint32)
    x2 = (x2 + ks[1]).astype(np.uint32)
    for i in range(5):
        for r in rot[i % 2]:
            x1 = (x1 + x2).astype(np.uint32)
            x2 = ((x2 << np.uint32(r)) | (x2 >> np.uint32(32 - r)))
            x2 = x2 ^ x1
        x1 = (x1 + ks[(i + 1) % 3]).astype(np.uint32)
        x2 = (x2 + ks[(i + 2) % 3] + np.uint32(i + 1)).astype(np.uint32)
    return x1, x2


def _iota_2x32(n):
    c = np.arange(n, dtype=np.uint64)
    return ((c >> np.uint64(32)).astype(np.uint32),
            (c & np.uint64(0xFFFFFFFF)).astype(np.uint32))


def _random_bits32(key, n):
    c1, c2 = _iota_2x32(n)
    b1, b2 = _threefry2x32(key[0], key[1], c1, c2)
    return b1 ^ b2


def _split2(key):
    c1, c2 = _iota_2x32(2)
    b1, b2 = _threefry2x32(key[0], key[1], c1, c2)
    return np.stack([b1, b2], axis=1)


def _permutation(seed, n):
    key = np.array([seed >> 32, seed & 0xFFFFFFFF], dtype=np.uint32)
    x = np.arange(n, dtype=np.int32)
    num_rounds = int(np.ceil(3 * np.log(max(1, n)) / np.log(2**32 - 1)))
    for _ in range(num_rounds):
        ks = _split2(key)
        key, subkey = ks[0], ks[1]
        sort_keys = _random_bits32(subkey, n)
        x = x[np.argsort(sort_keys, kind="stable")]
    return x


_SAMPLE_IDX = _permutation(42, _N)[:_S]

_GIDX = np.zeros((_NW * _CPW, _C), dtype=np.int32)
_GIDX[:_NCHUNKS] = (
    (np.arange(_B, dtype=np.int32) * _N)[:, None] + _SAMPLE_IDX[None, :]
).reshape(_NCHUNKS, _C)
_GIDX = _GIDX.reshape(_NW, _CPW, _C)

_P = np.arange(_XE, dtype=np.int64)
_XIDX = np.zeros((_NW * _XCPW, _XC), dtype=np.int32)
_XIDX[:_XNCH] = (
    (_P // (_B * _S)) * (_B * _N) + ((_P % (_B * _S)) // _S) * _N
    + _SAMPLE_IDX[_P % _S]
).astype(np.int32).reshape(_XNCH, _XC)
_XIDX = _XIDX.reshape(_NW, _XCPW, _XC)
del _P

_mesh = plsc.VectorSubcoreMesh(core_axis_name="c", subcore_axis_name="s")


@functools.partial(
    pl.kernel,
    out_type=[
        jax.ShapeDtypeStruct((_ROWS, _D), jnp.float32),
        jax.ShapeDtypeStruct((_XE,), jnp.float32),
    ],
    mesh=_mesh,
    scratch_types=[
        pltpu.VMEM((_CPW, _C), jnp.int32),
        pltpu.VMEM((_XCPW, _XC), jnp.int32),
        pltpu.VMEM((2, _K, _C, _D), jnp.float32),
        pltpu.VMEM((_XG * _XC,), jnp.float32),
        pltpu.VMEM((_XG * _XC,), jnp.float32),
        pltpu.SemaphoreType.DMA((2,)),
        pltpu.SemaphoreType.DMA((2,)),
        pltpu.SemaphoreType.DMA((2,)),
        pltpu.SemaphoreType.DMA((2,)),
    ],
)
def _gather(feat_hbm, xyz_hbm, gidx_hbm, xidx_hbm, feat_out, xyz_out,
            idx_v, xidx_v, fbuf, xbuf0, xbuf1, gsem, wsem, xgsem, xwsem):
    xbufs = (xbuf0, xbuf1)
    wid = lax.axis_index("s") * 2 + lax.axis_index("c")
    count = jnp.where(wid == _NW - 1, _LAST_COUNT, _CPW)
    xcount = jnp.where(wid == _NW - 1, _XLAST_COUNT, _XCPW)
    fobase = wid * _CPW
    xobase = wid * _XCPW * _XC
    pltpu.sync_copy(gidx_hbm.at[wid], idx_v)
    pltpu.sync_copy(xidx_hbm.at[wid], xidx_v)

    def fg_copy(t, bank, b):
        return pltpu.make_async_copy(
            feat_hbm.at[idx_v.at[t]], fbuf.at[bank].at[b], gsem.at[bank])

    def fw_copy(t, bank, b):
        return pltpu.make_async_copy(
            fbuf.at[bank].at[b],
            feat_out.at[pl.ds((fobase + t) * _C, _C)], wsem.at[bank])

    def xg_copy(xt, bank, b):
        return pltpu.make_async_copy(
            xyz_hbm.at[xidx_v.at[xt]],
            xbufs[bank].at[pl.ds(b * _XC, _XC)], xgsem.at[bank])

    def fire_fgathers(g, bank):
        for b in range(_K):
            t = g * _K + b

            @pl.when(t < count)
            def _():
                fg_copy(t, bank, b).start()

    def fire_xgathers(g, bank):
        for b in range(_XG):
            xt = g * _XG + b

            @pl.when(xt < xcount)
            def _():
                xg_copy(xt, bank, b).start()

    def xw_full(g, bank):
        return pltpu.make_async_copy(
            xbufs[bank],
            xyz_out.at[pl.ds(xobase + g * _XG * _XC, _XG * _XC)],
            xwsem.at[bank])

    def xw_part(xt, bank, b):
        return pltpu.make_async_copy(
            xbufs[bank].at[pl.ds(b * _XC, _XC)],
            xyz_out.at[pl.ds(xobase + xt * _XC, _XC)], xwsem.at[bank])

    def slot(g, p):
        q = 1 - p
        for b in range(_K):
            t = g * _K + b

            @pl.when(t < count)
            def _():
                fg_copy(t, p, b).wait()
        for b in range(_K):
            t = g * _K + b

            @pl.when(t < count)
            def _():
                fw_copy(t, p, b).start()
        for b in range(_K):
            tp = (g - 1) * _K + b

            @pl.when((g >= 1) & (tp < count))
            def _():
                fw_copy(tp, q, b).wait()
        fire_fgathers(g + 1, q)

        for b in range(_XG):
            xt = g * _XG + b

            @pl.when(xt < xcount)
            def _():
                xg_copy(xt, p, b).wait()
        g_full = (g * _XG + _XG) <= xcount
        g_any = (g * _XG) < xcount

        @pl.when(g_any & g_full)
        def _():
            xw_full(g, p).start()
        for b in range(_XG):
            xt = g * _XG + b

            @pl.when((~g_full) & (xt < xcount))
            def _():
                xw_part(xt, p, b).start()
        gp_full = ((g - 1) * _XG + _XG) <= xcount
        gp_any = (g >= 1) & (((g - 1) * _XG) < xcount)

        @pl.when(gp_any & gp_full)
        def _():
            xw_full(g - 1, q).wait()
        for b in range(_XG):
            xt = (g - 1) * _XG + b

            @pl.when(gp_any & (~gp_full) & (xt < xcount))
            def _():
                xw_part(xt, q, b).wait()
        fire_xgathers(g + 1, q)

    fire_fgathers(0, 0)
    fire_xgathers(0, 0)

    def pair(m, carry):
        slot(2 * m, 0)
        slot(2 * m + 1, 1)
        return carry

    lax.fori_loop(0, _NPAIR, pair, 0)


def kernel(xyz, features):
    feat_out, xyz_out = _gather(
        features.reshape(_B * _N, _D),
        jnp.transpose(xyz, (2, 0, 1)).reshape(_B * _N * 3),
        jnp.asarray(_GIDX),
        jnp.asarray(_XIDX),
    )
    new_xyz = jnp.transpose(xyz_out.reshape(3, _B, _S), (1, 2, 0))
    new_features = feat_out.reshape(_B, _S, _D)
    sample_idx_b = jnp.broadcast_to(jnp.asarray(_SAMPLE_IDX), (_B, _S))
    return (new_xyz, new_features, sample_idx_b)


# speedup vs baseline: 1.1296x; 1.0387x over previous
import functools

import numpy as np
import jax
import jax.numpy as jnp
from jax import lax
from jax.experimental import pallas as pl
from jax.experimental.pallas import tpu as pltpu
from jax.experimental.pallas import tpu_sc as plsc

_B = 8
_N = 100000
_S = 50000
_D = 128
_NW = 32
_ROWS = _B * _S

_C = 128
_K = 2
_NCHUNKS = _ROWS // _C
_CPW = -(-_NCHUNKS // _NW)
_LAST_COUNT = _NCHUNKS - (_NW - 1) * _CPW
_NG = -(-_CPW // _K)

_XC = 128
_XG = 4
_XE = _ROWS * 3
_XNCH = _XE // _XC
_XCPW = -(-_XNCH // _NW)
_XLAST_COUNT = _XNCH - (_NW - 1) * _XCPW
_NSLOTS = _NG + 1
_NPAIR = -(-_NSLOTS // 2)


def _threefry2x32(k1, k2, x1, x2):
    rot = [[13, 15, 26, 6], [17, 29, 16, 24]]
    ks = [k1, k2, (k1 ^ k2 ^ np.uint32(0x1BD11BDA))]
    x1 = (x1 + ks[0]).astype(np.uint32)
    x2 = (x2 + ks[1]).astype(np.uint32)
    for i in range(5):
        for r in rot[i % 2]:
            x1 = (x1 + x2).astype(np.uint32)
            x2 = ((x2 << np.uint32(r)) | (x2 >> np.uint32(32 - r)))
            x2 = x2 ^ x1
        x1 = (x1 + ks[(i + 1) % 3]).astype(np.uint32)
        x2 = (x2 + ks[(i + 2) % 3] + np.uint32(i + 1)).astype(np.uint32)
    return x1, x2


def _iota_2x32(n):
    c = np.arange(n, dtype=np.uint64)
    return ((c >> np.uint64(32)).astype(np.uint32),
            (c & np.uint64(0xFFFFFFFF)).astype(np.uint32))


def _random_bits32(key, n):
    c1, c2 = _iota_2x32(n)
    b1, b2 = _threefry2x32(key[0], key[1], c1, c2)
    return b1 ^ b2


def _split2(key):
    c1, c2 = _iota_2x32(2)
    b1, b2 = _threefry2x32(key[0], key[1], c1, c2)
    return np.stack([b1, b2], axis=1)


def _permutation(seed, n):
    key = np.array([seed >> 32, seed & 0xFFFFFFFF], dtype=np.uint32)
    x = np.arange(n, dtype=np.int32)
    num_rounds = int(np.ceil(3 * np.log(max(1, n)) / np.log(2**32 - 1)))
    for _ in range(num_rounds):
        ks = _split2(key)
        key, subkey = ks[0], ks[1]
        sort_keys = _random_bits32(subkey, n)
        x = x[np.argsort(sort_keys, kind="stable")]
    return x


_SAMPLE_IDX = _permutation(42, _N)[:_S]

_GIDX = np.zeros((_NW * _CPW, _C), dtype=np.int32)
_GIDX[:_NCHUNKS] = (
    (np.arange(_B, dtype=np.int32) * _N)[:, None] + _SAMPLE_IDX[None, :]
).reshape(_NCHUNKS, _C)
_GIDX = _GIDX.reshape(_NW, _CPW, _C)

_P = np.arange(_XE, dtype=np.int64)
_XIDX = np.zeros((_NW * _XCPW, _XC), dtype=np.int32)
_XIDX[:_XNCH] = (
    (_P // (_B * _S)) * (_B * _N) + ((_P % (_B * _S)) // _S) * _N
    + _SAMPLE_IDX[_P % _S]
).astype(np.int32).reshape(_XNCH, _XC)
_XIDX = _XIDX.reshape(_NW, _XCPW, _XC)
del _P

_mesh = plsc.VectorSubcoreMesh(core_axis_name="c", subcore_axis_name="s")


@functools.partial(
    pl.kernel,
    out_type=[
        jax.ShapeDtypeStruct((_ROWS, _D), jnp.float32),
        jax.ShapeDtypeStruct((_XE,), jnp.float32),
    ],
    mesh=_mesh,
    scratch_types=[
        pltpu.VMEM((_CPW, _C), jnp.int32),
        pltpu.VMEM((_XCPW, _XC), jnp.int32),
        pltpu.VMEM((2, _K, _C, _D), jnp.float32),
        pltpu.VMEM((_XG * _XC,), jnp.float32),
        pltpu.VMEM((_XG * _XC,), jnp.float32),
        pltpu.SemaphoreType.DMA((2,)),
        pltpu.SemaphoreType.DMA((2,)),
        pltpu.SemaphoreType.DMA((2,)),
        pltpu.SemaphoreType.DMA((2,)),
    ],
)
def _gather(feat_hbm, xyz_hbm, gidx_hbm, xidx_hbm, feat_out, xyz_out,
            idx_v, xidx_v, fbuf, xbuf0, xbuf1, gsem, wsem, xgsem, xwsem):
    xbufs = (xbuf0, xbuf1)
    wid = lax.axis_index("s") * 2 + lax.axis_index("c")
    count = jnp.where(wid == _NW - 1, _LAST_COUNT, _CPW)
    xcount = jnp.where(wid == _NW - 1, _XLAST_COUNT, _XCPW)
    fobase = wid * _CPW
    xobase = wid * _XCPW * _XC
    pltpu.sync_copy(gidx_hbm.at[wid], idx_v)
    pltpu.sync_copy(xidx_hbm.at[wid], xidx_v)

    def fg_copy(t, bank, b):
        return pltpu.make_async_copy(
            feat_hbm.at[idx_v.at[t]], fbuf.at[bank].at[b], gsem.at[bank])

    def fw_copy(t, bank, b):
        return pltpu.make_async_copy(
            fbuf.at[bank].at[b],
            feat_out.at[pl.ds((fobase + t) * _C, _C)], wsem.at[bank])

    def xg_copy(xt, bank, b):
        return pltpu.make_async_copy(
            xyz_hbm.at[xidx_v.at[xt]],
            xbufs[bank].at[pl.ds(b * _XC, _XC)], xgsem.at[bank])

    def fire_fgathers(g, bank):
        for b in range(_K):
            t = g * _K + b

            @pl.when(t < count)
            def _():
                fg_copy(t, bank, b).start()

    def fire_xgathers(g, bank):
        for b in range(_XG):
            xt = g * _XG + b

            @pl.when(xt < xcount)
            def _():
                xg_copy(xt, bank, b).start()

    def xw_full(g, bank):
        return pltpu.make_async_copy(
            xbufs[bank],
            xyz_out.at[pl.ds(xobase + g * _XG * _XC, _XG * _XC)],
            xwsem.at[bank])

    def xw_part(xt, bank, b):
        return pltpu.make_async_copy(
            xbufs[bank].at[pl.ds(b * _XC, _XC)],
            xyz_out.at[pl.ds(xobase + xt * _XC, _XC)], xwsem.at[bank])

    def slot(g, p):
        q = 1 - p
        for b in range(_K):
            t = g * _K + b

            @pl.when(t < count)
            def _():
                fg_copy(t, p, b).wait()
        for b in range(_K):
            t = g * _K + b

            @pl.when(t < count)
            def _():
                fw_copy(t, p, b).start()
        for b in range(_K):
            tp = (g - 1) * _K + b

            @pl.when((g >= 1) & (tp < count))
            def _():
                fw_copy(tp, q, b).wait()
        fire_fgathers(g + 1, q)

        for b in range(_XG):
            xt = g * _XG + b

            @pl.when(xt < xcount)
            def _():
                xg_copy(xt, p, b).wait()
        g_full = (g * _XG + _XG) <= xcount
        g_any = (g * _XG) < xcount

        @pl.when(g_any & g_full)
        def _():
            xw_full(g, p).start()
        for b in range(_XG):
            xt = g * _XG + b

            @pl.when((~g_full) & (xt < xcount))
            def _():
                xw_part(xt, p, b).start()
        gp_full = ((g - 1) * _XG + _XG) <= xcount
        gp_any = (g >= 1) & (((g - 1) * _XG) < xcount)

        @pl.when(gp_any & gp_full)
        def _():
            xw_full(g - 1, q).wait()
        for b in range(_XG):
            xt = (g - 1) * _XG + b

            @pl.when(gp_any & (~gp_full) & (xt < xcount))
            def _():
                xw_part(xt, q, b).wait()
        fire_xgathers(g + 1, q)

    fire_fgathers(0, 0)
    fire_xgathers(0, 0)

    def pair(m, carry):
        slot(2 * m, 0)
        slot(2 * m + 1, 1)
        return carry

    lax.fori_loop(0, _NPAIR, pair, 0)


def kernel(xyz, features):
    feat_out, xyz_out = _gather(
        features.reshape(_B * _N, _D),
        jnp.transpose(xyz, (2, 0, 1)).reshape(_B * _N * 3),
        jnp.asarray(_GIDX),
        jnp.asarray(_XIDX),
    )
    new_xyz = jnp.transpose(xyz_out.reshape(3, _B, _S), (1, 2, 0))
    new_features = feat_out.reshape(_B, _S, _D)
    sample_idx_b = jnp.broadcast_to(jnp.asarray(_SAMPLE_IDX), (_B, _S))
    return (new_xyz, new_features, sample_idx_b)
